# baseline (speedup 1.0000x reference)
import functools

import jax
import jax.numpy as jnp
from jax import lax
from jax.experimental import pallas as pl
from jax.experimental.pallas import tpu as pltpu
from jax.experimental.pallas import tpu_sc as plsc

_G = 4
_EPG = 4
_E = _G * _EPG
_EPS = 1e-9
_FF_T = 1024
_NR = 32


def _top2_of4(cols, zero, one, two, three):
    c0, c1, c2, c3 = cols
    m1 = jnp.maximum(jnp.maximum(c0, c1), jnp.maximum(c2, c3))
    i1 = jnp.where(c0 == m1, zero,
         jnp.where(c1 == m1, one,
         jnp.where(c2 == m1, two, three)))
    neg = jnp.float32(-jnp.inf)
    gids = (zero, one, two, three)
    masked = [jnp.where(i1 == gids[g], neg, cols[g]) for g in range(4)]
    m2 = jnp.maximum(jnp.maximum(masked[0], masked[1]),
                     jnp.maximum(masked[2], masked[3]))
    i2 = jnp.where(masked[0] == m2, zero,
         jnp.where(masked[1] == m2, one,
         jnp.where(masked[2] == m2, two, three)))
    denom = m1 + m2 + _EPS
    out = []
    for g in range(4):
        w = jnp.where(i1 == gids[g], m1,
            jnp.where(i2 == gids[g], m2, 0.0)) / denom
        out.append(w)
    return out


def _scores_kernel(x_ref, wr_ref, out_ref):
    out_ref[...] = jax.nn.sigmoid(
        jax.lax.dot_general(wr_ref[...], x_ref[...], (((1,), (1,)), ((), ())),
                            preferred_element_type=jnp.float32))


def _route_sc(probs_hbm, comb_hbm, probs_v, comb_v):
    cid = lax.axis_index("c")
    sid = lax.axis_index("s")

    @pl.when(jnp.logical_and(cid == 0, sid == 0))
    def _():
        pltpu.sync_copy(probs_hbm, probs_v)
        T = probs_hbm.shape[1]
        ids = [jnp.int32(v) for v in range(4)]
        for b in range(T // 16):
            sl = pl.ds(b * 16, 16)
            macro = [probs_v[g, sl] for g in range(_G)]
            mw = _top2_of4(macro, *ids)
            for g in range(_G):
                micro = [probs_v[_G + g * _EPG + j, sl] for j in range(_EPG)]
                uw = _top2_of4(micro, *ids)
                for j in range(_EPG):
                    comb_v[g * _EPG + j, sl] = mw[g] * uw[j]
        pltpu.sync_copy(comb_v, comb_hbm)


def _moe_kernel(x_ref, comb_ref, wg_ref, wu_ref, wd_ref, out_ref):
    e = pl.program_id(0)
    f = pl.program_id(1)

    xf = x_ref[...]
    wg = wg_ref[0]
    wu = wu_ref[0]
    wd = wd_ref[0]
    dn = (((1,), (1,)), ((), ()))
    g1 = jax.lax.dot_general(xf, wg, dn, preferred_element_type=jnp.float32)
    up = jax.lax.dot_general(xf, wu, dn, preferred_element_type=jnp.float32)
    h = (g1 * jax.nn.sigmoid(g1)) * up
    h = h * comb_ref[e]
    part = jax.lax.dot_general(h, wd, dn, preferred_element_type=jnp.float32)

    @pl.when(jnp.logical_and(e == 0, f == 0))
    def _init():
        out_ref[...] = part

    @pl.when(jnp.logical_not(jnp.logical_and(e == 0, f == 0)))
    def _acc():
        out_ref[...] += part


def kernel(x, macro_w, micro_w, w_gate, w_up, w_down):
    bsz, seq_len, d_model = x.shape
    T = bsz * seq_len
    E, FF, D = w_gate.shape
    xf = x.reshape(T, d_model)

    wr = jnp.concatenate([macro_w, micro_w.reshape(E, D)], axis=0)
    wr = jnp.pad(wr, ((0, _NR - _G - E), (0, 0)))

    probs = pl.pallas_call(
        _scores_kernel,
        out_shape=jax.ShapeDtypeStruct((_NR, T), jnp.float32),
    )(xf, wr)

    route = functools.partial(
        pl.kernel,
        out_type=jax.ShapeDtypeStruct((_E, T), jnp.float32),
        mesh=plsc.VectorSubcoreMesh(core_axis_name="c", subcore_axis_name="s"),
        scratch_types=[
            pltpu.VMEM((_NR, T), jnp.float32),
            pltpu.VMEM((_E, T), jnp.float32),
        ],
    )(_route_sc)
    comb = route(probs)

    nf = FF // _FF_T
    out = pl.pallas_call(
        _moe_kernel,
        grid=(E, nf),
        in_specs=[
            pl.BlockSpec((T, D), lambda e, f: (0, 0)),
            pl.BlockSpec((_E, T, 1), lambda e, f: (0, 0, 0)),
            pl.BlockSpec((1, _FF_T, D), lambda e, f: (e, f, 0)),
            pl.BlockSpec((1, _FF_T, D), lambda e, f: (e, f, 0)),
            pl.BlockSpec((1, D, _FF_T), lambda e, f: (e, 0, f)),
        ],
        out_specs=pl.BlockSpec((T, D), lambda e, f: (0, 0)),
        out_shape=jax.ShapeDtypeStruct((T, D), jnp.float32),
        compiler_params=pltpu.CompilerParams(
            dimension_semantics=("arbitrary", "arbitrary")),
    )(xf, comb.reshape(_E, T, 1), w_gate, w_up, w_down)
    return out.reshape(bsz, seq_len, d_model)


# speedup vs baseline: 1.0191x; 1.0191x over previous
import functools

import jax
import jax.numpy as jnp
from jax import lax
from jax.experimental import pallas as pl
from jax.experimental.pallas import tpu as pltpu
from jax.experimental.pallas import tpu_sc as plsc

_G = 4
_EPG = 4
_E = _G * _EPG
_EPS = 1e-9
_FF_T = 1024
_NR = 32


def _top2_of4(cols, zero, one, two, three):
    c0, c1, c2, c3 = cols
    m1 = jnp.maximum(jnp.maximum(c0, c1), jnp.maximum(c2, c3))
    i1 = jnp.where(c0 == m1, zero,
         jnp.where(c1 == m1, one,
         jnp.where(c2 == m1, two, three)))
    neg = jnp.float32(-jnp.inf)
    gids = (zero, one, two, three)
    masked = [jnp.where(i1 == gids[g], neg, cols[g]) for g in range(4)]
    m2 = jnp.maximum(jnp.maximum(masked[0], masked[1]),
                     jnp.maximum(masked[2], masked[3]))
    i2 = jnp.where(masked[0] == m2, zero,
         jnp.where(masked[1] == m2, one,
         jnp.where(masked[2] == m2, two, three)))
    denom = m1 + m2 + _EPS
    out = []
    for g in range(4):
        w = jnp.where(i1 == gids[g], m1,
            jnp.where(i2 == gids[g], m2, 0.0)) / denom
        out.append(w)
    return out


def _scores_kernel(x_ref, wr_ref, out_ref):
    out_ref[...] = jax.nn.sigmoid(
        jax.lax.dot_general(wr_ref[...], x_ref[...], (((1,), (1,)), ((), ())),
                            preferred_element_type=jnp.float32))


def _route_sc(probs_hbm, comb_hbm, probs_v, comb_v):
    cid = lax.axis_index("c")
    sid = lax.axis_index("s")

    @pl.when(jnp.logical_and(cid == 0, sid == 0))
    def _():
        pltpu.sync_copy(probs_hbm, probs_v)
        T = probs_hbm.shape[1]
        ids = [jnp.int32(v) for v in range(4)]
        for b in range(T // 16):
            sl = pl.ds(b * 16, 16)
            macro = [probs_v[g, sl] for g in range(_G)]
            mw = _top2_of4(macro, *ids)
            for g in range(_G):
                micro = [probs_v[_G + g * _EPG + j, sl] for j in range(_EPG)]
                uw = _top2_of4(micro, *ids)
                for j in range(_EPG):
                    comb_v[g * _EPG + j, sl] = mw[g] * uw[j]
        pltpu.sync_copy(comb_v, comb_hbm)


def _moe_kernel(x_ref, comb_ref, wg_ref, wu_ref, wd_ref, out_ref, c3_ref):
    e = pl.program_id(0)
    f = pl.program_id(1)

    @pl.when(jnp.logical_and(e == 0, f == 0))
    def _stage_comb():
        combT = jnp.transpose(comb_ref[...], (1, 0))
        for ee in range(_E):
            c3_ref[ee] = combT[:, ee:ee + 1]

    xf = x_ref[...]
    wg = wg_ref[0]
    wu = wu_ref[0]
    wd = wd_ref[0]
    dn = (((1,), (1,)), ((), ()))
    g1 = jax.lax.dot_general(xf, wg, dn, preferred_element_type=jnp.float32)
    up = jax.lax.dot_general(xf, wu, dn, preferred_element_type=jnp.float32)
    h = (g1 * jax.nn.sigmoid(g1)) * up
    h = h * c3_ref[e]
    part = jax.lax.dot_general(h, wd, dn, preferred_element_type=jnp.float32)

    @pl.when(jnp.logical_and(e == 0, f == 0))
    def _init():
        out_ref[...] = part

    @pl.when(jnp.logical_not(jnp.logical_and(e == 0, f == 0)))
    def _acc():
        out_ref[...] += part


def kernel(x, macro_w, micro_w, w_gate, w_up, w_down):
    bsz, seq_len, d_model = x.shape
    T = bsz * seq_len
    E, FF, D = w_gate.shape
    xf = x.reshape(T, d_model)

    wr = jnp.concatenate([macro_w, micro_w.reshape(E, D)], axis=0)
    wr = jnp.pad(wr, ((0, _NR - _G - E), (0, 0)))

    probs = pl.pallas_call(
        _scores_kernel,
        out_shape=jax.ShapeDtypeStruct((_NR, T), jnp.float32),
    )(xf, wr)

    route = functools.partial(
        pl.kernel,
        out_type=jax.ShapeDtypeStruct((_E, T), jnp.float32),
        mesh=plsc.VectorSubcoreMesh(core_axis_name="c", subcore_axis_name="s"),
        scratch_types=[
            pltpu.VMEM((_NR, T), jnp.float32),
            pltpu.VMEM((_E, T), jnp.float32),
        ],
    )(_route_sc)
    comb = route(probs)

    nf = FF // _FF_T
    out = pl.pallas_call(
        _moe_kernel,
        grid=(E, nf),
        in_specs=[
            pl.BlockSpec((T, D), lambda e, f: (0, 0)),
            pl.BlockSpec((_E, T), lambda e, f: (0, 0)),
            pl.BlockSpec((1, _FF_T, D), lambda e, f: (e, f, 0)),
            pl.BlockSpec((1, _FF_T, D), lambda e, f: (e, f, 0)),
            pl.BlockSpec((1, D, _FF_T), lambda e, f: (e, 0, f)),
        ],
        out_specs=pl.BlockSpec((T, D), lambda e, f: (0, 0)),
        out_shape=jax.ShapeDtypeStruct((T, D), jnp.float32),
        scratch_shapes=[pltpu.VMEM((_E, T, 1), jnp.float32)],
        compiler_params=pltpu.CompilerParams(
            dimension_semantics=("arbitrary", "arbitrary")),
    )(xf, comb, w_gate, w_up, w_down)
    return out.reshape(bsz, seq_len, d_model)
